# baseline (speedup 1.0000x reference)
import functools

import jax
import jax.numpy as jnp
from jax import lax
from jax.experimental import pallas as pl
from jax.experimental.pallas import tpu as pltpu
from jax.experimental.pallas import tpu_sc as plsc

NUM_EMB = 1000000
EMBED_DIM = 32
BATCH = 16384
N_FIELDS = 26

NUM_CORES = 2
NUM_SUBCORES = 16
NUM_WORKERS = NUM_CORES * NUM_SUBCORES
BB = 128
N_BLOCKS = N_FIELDS * (BATCH // BB)
BLOCKS_PER_W = N_BLOCKS // NUM_WORKERS

_mesh = plsc.VectorSubcoreMesh(core_axis_name="c", subcore_axis_name="s")


def _transpose_block(rows_v, obuf):
    lane = lax.iota(jnp.int32, 16)
    for db in range(4):
        for s in range(8):
            d = db * 8 + s
            col = jnp.full((16,), d, jnp.int32)
            for lg in range(8):
                row_i = lane + (lg * 16)
                v = plsc.load_gather(rows_v, [row_i, col])
                obuf[db, s, pl.ds(lg * 16, 16)] = v


@functools.partial(
    pl.kernel,
    mesh=_mesh,
    out_type=jax.ShapeDtypeStruct((N_BLOCKS * 4, 8, BB), jnp.float32),
    scratch_types=[
        pltpu.VMEM((BB,), jnp.int32),
        pltpu.VMEM((BB,), jnp.int32),
        pltpu.VMEM((BB, EMBED_DIM), jnp.float32),
        pltpu.VMEM((BB, EMBED_DIM), jnp.float32),
        pltpu.VMEM((4, 8, BB), jnp.float32),
        pltpu.VMEM((4, 8, BB), jnp.float32),
        pltpu.SemaphoreType.DMA,
        pltpu.SemaphoreType.DMA,
        pltpu.SemaphoreType.DMA,
        pltpu.SemaphoreType.DMA,
    ],
    compiler_params=pltpu.CompilerParams(use_tc_tiling_on_sc=False,
                                         needs_layout_passes=False),
)
def _emb_lookup(idx_hbm, table_hbm, out_hbm, idx0, idx1, rows0, rows1,
                ob0, ob1, g0, g1, w0, w1):
    wid = lax.axis_index("s") * NUM_CORES + lax.axis_index("c")
    g_base = wid * BLOCKS_PER_W

    def body(half, _):
        k0 = g_base + 2 * half
        handles = []
        for (koff, idxv, rows, gsem) in ((0, idx0, rows0, g0),
                                         (1, idx1, rows1, g1)):
            g = k0 + koff
            f = g // (BATCH // BB)
            bb = g % (BATCH // BB)
            pltpu.sync_copy(idx_hbm.at[f, pl.ds(bb * BB, BB)], idxv)
            handles.append(pltpu.async_copy(table_hbm.at[idxv], rows, gsem))
        wr = []
        for (koff, rows, obuf, gh, wsem) in ((0, rows0, ob0, handles[0], w0),
                                             (1, rows1, ob1, handles[1], w1)):
            g = k0 + koff
            f = g // (BATCH // BB)
            bb = g % (BATCH // BB)
            gh.wait()
            _transpose_block(rows, obuf)
            for db in range(4):
                r = (f * 4 + db) * BB + bb
                wr.append(pltpu.async_copy(obuf.at[db], out_hbm.at[r], wsem))
        for h in wr:
            h.wait()
        return _

    lax.fori_loop(0, BLOCKS_PER_W // 2, body, None)


def kernel(batch_cat, weight):
    idx_t = batch_cat.T.astype(jnp.int32)
    out3 = _emb_lookup(idx_t, weight)
    out = (out3.reshape(N_FIELDS, 4, BATCH // BB, 8, BB)
           .transpose(2, 4, 0, 1, 3)
           .reshape(BATCH, N_FIELDS, EMBED_DIM))
    return out


# speedup vs baseline: 1.1519x; 1.1519x over previous
import functools

import jax
import jax.numpy as jnp
from jax import lax
from jax.experimental import pallas as pl
from jax.experimental.pallas import tpu as pltpu
from jax.experimental.pallas import tpu_sc as plsc

NUM_EMB = 1000000
EMBED_DIM = 32
BATCH = 16384
N_FIELDS = 26

NUM_CORES = 2
NUM_SUBCORES = 16
NUM_WORKERS = NUM_CORES * NUM_SUBCORES
BB = 128
NBB = BATCH // BB
N_BLOCKS = N_FIELDS * NBB
BLOCKS_PER_W = N_BLOCKS // NUM_WORKERS

_mesh = plsc.VectorSubcoreMesh(core_axis_name="c", subcore_axis_name="s")


def _transpose_block(rows_v, obuf):
    lane = lax.iota(jnp.int32, 16)

    @plsc.parallel_loop(0, 256, 1, unroll=8)
    def _(t):
        d = t // 8
        lg = t % 8
        row_i = lane + lg * 16
        col = jnp.full((16,), d, jnp.int32)
        obuf[pl.ds(t * 16, 16)] = plsc.load_gather(rows_v, [row_i, col])


@functools.partial(
    pl.kernel,
    mesh=_mesh,
    out_type=jax.ShapeDtypeStruct((N_BLOCKS * 4, 8 * BB), jnp.float32),
    scratch_types=[
        pltpu.VMEM((BB,), jnp.int32),
        pltpu.VMEM((BB,), jnp.int32),
        pltpu.VMEM((BB, EMBED_DIM), jnp.float32),
        pltpu.VMEM((BB, EMBED_DIM), jnp.float32),
        pltpu.VMEM((4 * 8 * BB,), jnp.float32),
        pltpu.VMEM((4 * 8 * BB,), jnp.float32),
        pltpu.SemaphoreType.DMA,
        pltpu.SemaphoreType.DMA,
        pltpu.SemaphoreType.DMA,
        pltpu.SemaphoreType.DMA,
    ],
    compiler_params=pltpu.CompilerParams(use_tc_tiling_on_sc=False,
                                         needs_layout_passes=False),
)
def _emb_lookup(idx_hbm, table_hbm, out_hbm, idx0, idx1, rows0, rows1,
                ob0, ob1, g0, g1, w0, w1):
    wid = lax.axis_index("s") * NUM_CORES + lax.axis_index("c")
    g_base = wid * BLOCKS_PER_W

    def body(half, carry):
        k0 = g_base + 2 * half
        handles = []
        for (koff, idxv, rows, gsem) in ((0, idx0, rows0, g0),
                                         (1, idx1, rows1, g1)):
            g = k0 + koff
            f = g // NBB
            bb = g % NBB
            pltpu.sync_copy(idx_hbm.at[f, pl.ds(bb * BB, BB)], idxv)
            handles.append(pltpu.async_copy(table_hbm.at[idxv], rows, gsem))
        wr = []
        for (koff, rows, obuf, gh, wsem) in ((0, rows0, ob0, handles[0], w0),
                                             (1, rows1, ob1, handles[1], w1)):
            g = k0 + koff
            f = g // NBB
            bb = g % NBB
            gh.wait()
            _transpose_block(rows, obuf)
            for db in range(4):
                r = (f * 4 + db) * BB + bb
                wr.append(pltpu.async_copy(obuf.at[pl.ds(db * 1024, 1024)],
                                           out_hbm.at[r], wsem))
        for h in wr:
            h.wait()
        return carry

    lax.fori_loop(0, BLOCKS_PER_W // 2, body, 0)


def kernel(batch_cat, weight):
    idx_t = batch_cat.T.astype(jnp.int32)
    out3 = _emb_lookup(idx_t, weight)
    out = (out3.reshape(N_FIELDS, 4, NBB, 8, BB)
           .transpose(2, 4, 0, 1, 3)
           .reshape(BATCH, N_FIELDS, EMBED_DIM))
    return out


# speedup vs baseline: 1.5090x; 1.3100x over previous
import functools

import jax
import jax.numpy as jnp
from jax import lax
from jax.experimental import pallas as pl
from jax.experimental.pallas import tpu as pltpu
from jax.experimental.pallas import tpu_sc as plsc

NUM_EMB = 1000000
EMBED_DIM = 32
BATCH = 16384
N_FIELDS = 26

NUM_CORES = 2
NUM_SUBCORES = 16
NUM_WORKERS = NUM_CORES * NUM_SUBCORES
BB = 128
NBB = BATCH // BB
N_BLOCKS = N_FIELDS * NBB
BLOCKS_PER_W = N_BLOCKS // NUM_WORKERS

_mesh = plsc.VectorSubcoreMesh(core_axis_name="c", subcore_axis_name="s")


OB_STRIDE = 129


def _transpose_block(rows_v, obuf):
    lane = lax.iota(jnp.int32, 16)
    hi = lane + 16

    @plsc.parallel_loop(0, BB, 1, unroll=8)
    def _(l):
        v0 = rows_v[l, pl.ds(0, 16)]
        v1 = rows_v[l, pl.ds(16, 16)]
        col = jnp.full((16,), l, jnp.int32)
        plsc.store_scatter(obuf, [lane, col], v0)
        plsc.store_scatter(obuf, [hi, col], v1)


@functools.partial(
    pl.kernel,
    mesh=_mesh,
    out_type=jax.ShapeDtypeStruct((N_BLOCKS * 4, 8, BB), jnp.float32),
    scratch_types=[
        pltpu.VMEM((BB,), jnp.int32),
        pltpu.VMEM((BB,), jnp.int32),
        pltpu.VMEM((BB, EMBED_DIM), jnp.float32),
        pltpu.VMEM((BB, EMBED_DIM), jnp.float32),
        pltpu.VMEM((EMBED_DIM, OB_STRIDE), jnp.float32),
        pltpu.VMEM((EMBED_DIM, OB_STRIDE), jnp.float32),
        pltpu.SemaphoreType.DMA,
        pltpu.SemaphoreType.DMA,
        pltpu.SemaphoreType.DMA,
        pltpu.SemaphoreType.DMA,
    ],
    compiler_params=pltpu.CompilerParams(use_tc_tiling_on_sc=False,
                                         needs_layout_passes=False),
)
def _emb_lookup(idx_hbm, table_hbm, out_hbm, idx0, idx1, rows0, rows1,
                ob0, ob1, g0, g1, w0, w1):
    wid = lax.axis_index("s") * NUM_CORES + lax.axis_index("c")
    g_base = wid * BLOCKS_PER_W

    def body(half, carry):
        k0 = g_base + 2 * half
        handles = []
        for (koff, idxv, rows, gsem) in ((0, idx0, rows0, g0),
                                         (1, idx1, rows1, g1)):
            g = k0 + koff
            f = g // NBB
            bb = g % NBB
            pltpu.sync_copy(idx_hbm.at[f, pl.ds(bb * BB, BB)], idxv)
            handles.append(pltpu.async_copy(table_hbm.at[idxv], rows, gsem))
        wr = []
        for (koff, rows, obuf, gh, wsem) in ((0, rows0, ob0, handles[0], w0),
                                             (1, rows1, ob1, handles[1], w1)):
            g = k0 + koff
            f = g // NBB
            bb = g % NBB
            gh.wait()
            _transpose_block(rows, obuf)
            for db in range(4):
                r = (f * 4 + db) * BB + bb
                wr.append(pltpu.async_copy(
                    obuf.at[pl.ds(db * 8, 8), pl.ds(0, BB)],
                    out_hbm.at[r], wsem))
        for h in wr:
            h.wait()
        return carry

    lax.fori_loop(0, BLOCKS_PER_W // 2, body, 0)


def kernel(batch_cat, weight):
    idx_t = batch_cat.T.astype(jnp.int32)
    out3 = _emb_lookup(idx_t, weight)
    out = (out3.reshape(N_FIELDS, 4, NBB, 8, BB)
           .transpose(2, 4, 0, 1, 3)
           .reshape(BATCH, N_FIELDS, EMBED_DIM))
    return out
